# Initial kernel scaffold; baseline (speedup 1.0000x reference)
import jax
import jax.numpy as jnp
from jax.experimental import pallas as pl


def kernel(x, edge_index, W1, b1, W2, b2, W3, b3):
    raise NotImplementedError("write your pallas kernel here")


# speedup vs baseline: 30.9270x; 30.9270x over previous
import functools

import jax
import jax.numpy as jnp
from jax import lax
from jax.experimental import pallas as pl
from jax.experimental.pallas import tpu as pltpu
from jax.experimental.pallas import tpu_sc as plsc

N = 10000
NPAD = 10240
E = 320000
D_IN = 128
H = 16
NCORES = 2
NSUB = 16
NW = NCORES * NSUB
EB = 128
ECH = 79
EPT = EB * ECH
EPAD = EPT * NW
DEG_EPT = E // NW
NROWS = NPAD // H
ROWS_PER_TILE = NROWS // NSUB

_mesh = plsc.VectorSubcoreMesh(core_axis_name="c", subcore_axis_name="s",
                               num_cores=NCORES, num_subcores=NSUB)


def _sc_deg_body(dst_hbm, deg_hbm, ldeg, dstb):
    c = lax.axis_index("c")
    s = lax.axis_index("s")
    t = c * NSUB + s

    zero16 = jnp.zeros((H,), jnp.float32)

    def _zero(i, _):
        ldeg[pl.ds(i * H, H)] = zero16
        return 0
    lax.fori_loop(0, NPAD // H, _zero, 0)

    pltpu.sync_copy(dst_hbm.at[t], dstb)

    ones16 = jnp.ones((H,), jnp.float32)

    def _hist(i, _):
        idx = dstb[pl.ds(i * H, H)]
        plsc.addupdate_scatter(ldeg, [idx], ones16)
        return 0
    lax.fori_loop(0, DEG_EPT // H, _hist, 0)

    pltpu.sync_copy(ldeg, deg_hbm.at[t])


def _make_sc_deg():
    return pl.kernel(
        _sc_deg_body,
        out_type=jax.ShapeDtypeStruct((NW, NPAD), jnp.float32),
        mesh=_mesh,
        scratch_types=[
            pltpu.VMEM((NPAD,), jnp.float32),
            pltpu.VMEM((DEG_EPT,), jnp.int32),
        ],
        compiler_params=pltpu.CompilerParams(needs_layout_passes=False,
                                             use_tc_tiling_on_sc=False),
    )


def _sc_agg_kernel(g_hbm, src_hbm, dst_hbm, agg_hbm, acc, srcb, dstb, rows,
                   obuf, sem):
    c = lax.axis_index("c")
    s = lax.axis_index("s")
    t = c * NSUB + s
    rows_per_sub = NPAD // NSUB

    zero16 = jnp.zeros((H,), jnp.float32)

    def _zero(i, _):
        obuf[i] = zero16
        return 0
    lax.fori_loop(0, rows_per_sub, _zero, 0)
    pltpu.sync_copy(obuf, acc.at[pl.ds(rows_per_sub * s, rows_per_sub)])

    pltpu.sync_copy(src_hbm.at[t], srcb)
    pltpu.sync_copy(dst_hbm.at[t], dstb)
    plsc.subcore_barrier()

    def _chunk(j, _):
        pltpu.async_copy(g_hbm.at[srcb.at[j]], rows, sem).wait()
        pltpu.sync_copy(rows, acc.at[dstb.at[j]], add=True)
        return 0
    lax.fori_loop(0, ECH, _chunk, 0)

    plsc.subcore_barrier()
    pltpu.sync_copy(acc.at[pl.ds(rows_per_sub * s, rows_per_sub)], obuf)
    pltpu.sync_copy(obuf, agg_hbm.at[c].at[pl.ds(rows_per_sub * s, rows_per_sub)])


def _make_sc_agg():
    return pl.kernel(
        _sc_agg_kernel,
        out_type=jax.ShapeDtypeStruct((NCORES, NPAD, H), jnp.float32),
        mesh=_mesh,
        scratch_types=[
            pltpu.VMEM_SHARED((NPAD, H), jnp.float32),
            pltpu.VMEM((ECH, EB), jnp.int32),
            pltpu.VMEM((ECH, EB), jnp.int32),
            pltpu.VMEM((EB, H), jnp.float32),
            pltpu.VMEM((NPAD // NSUB, H), jnp.float32),
            pltpu.SemaphoreType.DMA,
        ],
        compiler_params=pltpu.CompilerParams(needs_layout_passes=False,
                                             use_tc_tiling_on_sc=False),
    )


_RB = 1024


def _tcdeg_body(parts_ref, dinv_ref):
    deg = jnp.sum(parts_ref[...], axis=0, keepdims=True) + 1.0
    dinv_ref[...] = lax.rsqrt(deg)


def _tcdeg(deg_parts):
    return pl.pallas_call(
        _tcdeg_body,
        grid=(_GRID,),
        in_specs=[pl.BlockSpec((NW, _RB), lambda i: (0, i))],
        out_specs=pl.BlockSpec((1, _RB), lambda i: (0, i)),
        out_shape=jax.ShapeDtypeStruct((1, NPAD), jnp.float32),
    )(deg_parts)


def _tc1_body(x_ref, w_ref, dinv_ref, g_ref):
    h = jnp.dot(x_ref[...], w_ref[...], preferred_element_type=jnp.float32)
    g_ref[...] = h * dinv_ref[...]


def _tc_mid_body(a0_ref, a1_ref, g_ref, dinv_ref, b_ref, w_ref, out_ref):
    a = (a0_ref[...] + a1_ref[...] + g_ref[...]) * dinv_ref[...] + b_ref[...]
    r = jnp.maximum(a, 0.0)
    h = jnp.dot(r, w_ref[...], preferred_element_type=jnp.float32)
    out_ref[...] = h * dinv_ref[...]


def _tc_out_body(a0_ref, a1_ref, g_ref, dinv_ref, b_ref, w_ref, b3_ref, out_ref):
    a = (a0_ref[...] + a1_ref[...] + g_ref[...]) * dinv_ref[...] + b_ref[...]
    r = jnp.maximum(a, 0.0)
    out_ref[...] = jnp.dot(r, w_ref[...],
                           preferred_element_type=jnp.float32) + b3_ref[...]


def _row_spec(width):
    return pl.BlockSpec((_RB, width), lambda i: (i, 0))


def _full_spec(shape):
    return pl.BlockSpec(shape, lambda i: tuple(0 for _ in shape))


_GRID = NPAD // _RB


def _tc1(xp, W1, dinv_col):
    return pl.pallas_call(
        _tc1_body,
        grid=(_GRID,),
        in_specs=[_row_spec(D_IN), _full_spec((D_IN, H)), _row_spec(1)],
        out_specs=_row_spec(H),
        out_shape=jax.ShapeDtypeStruct((NPAD, H), jnp.float32),
    )(xp, W1, dinv_col)


def _tc_mid(a0, a1, g, dinv_col, b_row, W):
    return pl.pallas_call(
        _tc_mid_body,
        grid=(_GRID,),
        in_specs=[_row_spec(H), _row_spec(H), _row_spec(H), _row_spec(1),
                  _full_spec((1, H)), _full_spec((H, H))],
        out_specs=_row_spec(H),
        out_shape=jax.ShapeDtypeStruct((NPAD, H), jnp.float32),
    )(a0, a1, g, dinv_col, b_row, W)


def _tc_out(a0, a1, g, dinv_col, b_row, W3p, b3_row):
    return pl.pallas_call(
        _tc_out_body,
        grid=(_GRID,),
        in_specs=[_row_spec(H), _row_spec(H), _row_spec(H), _row_spec(1),
                  _full_spec((1, H)), _full_spec((H, 8)), _full_spec((1, 8))],
        out_specs=_row_spec(8),
        out_shape=jax.ShapeDtypeStruct((NPAD, 8), jnp.float32),
    )(a0, a1, g, dinv_col, b_row, W3p, b3_row)


@jax.jit
def _run(x, src, dst, W1, b1, W2, b2, W3, b3):
    src = src.astype(jnp.int32)
    dst = dst.astype(jnp.int32)

    pad = EPAD - E
    srcp = jnp.concatenate([src, jnp.zeros((pad,), jnp.int32)]).reshape(NW, ECH, EB)
    dstp = jnp.concatenate(
        [dst, N + (jnp.arange(pad, dtype=jnp.int32) % (NPAD - N))]
    ).reshape(NW, ECH, EB)
    dst_deg = dst.reshape(NW, DEG_EPT)

    xp = jnp.zeros((NPAD, D_IN), jnp.float32).at[:N].set(x)

    deg_parts = _make_sc_deg()(dst_deg)
    dinv_col = _tcdeg(deg_parts).reshape(NPAD, 1)

    g1 = _tc1(xp, W1, dinv_col)

    agg_fn = _make_sc_agg()
    a1 = agg_fn(g1, srcp, dstp)
    g2 = _tc_mid(a1[0], a1[1], g1, dinv_col, b1.reshape(1, H), W2)
    a2 = agg_fn(g2, srcp, dstp)
    out = _tc_out(a2[0], a2[1], g2, dinv_col, b2.reshape(1, H),
                  jnp.zeros((H, 8), jnp.float32).at[:, :7].set(W3),
                  jnp.zeros((1, 8), jnp.float32).at[0, :7].set(b3))
    return out[:N, :7]


def kernel(x, edge_index, W1, b1, W2, b2, W3, b3):
    return _run(x, edge_index[0], edge_index[1], W1, b1, W2, b2, W3, b3)


# speedup vs baseline: 39.1570x; 1.2661x over previous
import jax
import jax.numpy as jnp
from jax import lax
from jax.experimental import pallas as pl
from jax.experimental.pallas import tpu as pltpu
from jax.experimental.pallas import tpu_sc as plsc

N = 10000
NPAD = 10240
E = 320000
D_IN = 128
H = 16
NCORES = 2
NSUB = 16
NW = NCORES * NSUB
EB = 128
ECH = 79
EPT = EB * ECH
EPAD = EPT * NW

_mesh = plsc.VectorSubcoreMesh(core_axis_name="c", subcore_axis_name="s",
                               num_cores=NCORES, num_subcores=NSUB)

_sc_params = pltpu.CompilerParams(needs_layout_passes=False,
                                  use_tc_tiling_on_sc=False)


def _sc_deg_body(dst_hbm, deg_hbm, ldeg, dstb):
    c = lax.axis_index("c")
    s = lax.axis_index("s")
    t = c * NSUB + s

    zero16 = jnp.zeros((H,), jnp.float32)

    def _zero(i, _):
        ldeg[pl.ds(i * H, H)] = zero16
        return 0
    lax.fori_loop(0, NPAD // H, _zero, 0)

    pltpu.sync_copy(dst_hbm.at[t], dstb)

    ones16 = jnp.ones((H,), jnp.float32)

    def _hist(r, _):
        for k in range(EB // H):
            idx = dstb[r, pl.ds(k * H, H)]
            plsc.addupdate_scatter(ldeg, [idx], ones16)
        return 0
    lax.fori_loop(0, ECH, _hist, 0)

    pltpu.sync_copy(ldeg, deg_hbm.at[t])


def _make_sc_deg():
    return pl.kernel(
        _sc_deg_body,
        out_type=jax.ShapeDtypeStruct((NW, NPAD), jnp.float32),
        mesh=_mesh,
        scratch_types=[
            pltpu.VMEM((NPAD,), jnp.float32),
            pltpu.VMEM((ECH, EB), jnp.int32),
        ],
        compiler_params=_sc_params,
    )


def _sc_agg_kernel(g_hbm, src_hbm, dst_hbm, agg_hbm, acc, srcb, dstb,
                   rows_a, rows_b, obuf, sem_a, sem_b):
    c = lax.axis_index("c")
    s = lax.axis_index("s")
    t = c * NSUB + s
    rows_per_sub = NPAD // NSUB

    zero16 = jnp.zeros((H,), jnp.float32)

    def _zero(i, _):
        obuf[i] = zero16
        return 0
    lax.fori_loop(0, rows_per_sub, _zero, 0)
    pltpu.sync_copy(obuf, acc.at[pl.ds(rows_per_sub * s, rows_per_sub)])

    pltpu.sync_copy(src_hbm.at[t], srcb)
    pltpu.sync_copy(dst_hbm.at[t], dstb)
    plsc.subcore_barrier()

    def _wait(buf, sem):
        pltpu.make_async_copy(g_hbm.at[pl.ds(0, EB)], buf, sem).wait()

    pltpu.async_copy(g_hbm.at[srcb.at[0]], rows_a, sem_a)

    def _pair(i, _):
        j = 2 * i
        pltpu.async_copy(g_hbm.at[srcb.at[j + 1]], rows_b, sem_b)
        _wait(rows_a, sem_a)
        pltpu.sync_copy(rows_a, acc.at[dstb.at[j]], add=True)
        pltpu.async_copy(g_hbm.at[srcb.at[j + 2]], rows_a, sem_a)
        _wait(rows_b, sem_b)
        pltpu.sync_copy(rows_b, acc.at[dstb.at[j + 1]], add=True)
        return 0
    lax.fori_loop(0, (ECH - 1) // 2, _pair, 0)

    _wait(rows_a, sem_a)
    pltpu.sync_copy(rows_a, acc.at[dstb.at[ECH - 1]], add=True)

    plsc.subcore_barrier()
    pltpu.sync_copy(acc.at[pl.ds(rows_per_sub * s, rows_per_sub)], obuf)
    pltpu.sync_copy(obuf, agg_hbm.at[c].at[pl.ds(rows_per_sub * s, rows_per_sub)])


def _make_sc_agg():
    return pl.kernel(
        _sc_agg_kernel,
        out_type=jax.ShapeDtypeStruct((NCORES, NPAD, H), jnp.float32),
        mesh=_mesh,
        scratch_types=[
            pltpu.VMEM_SHARED((NPAD, H), jnp.float32),
            pltpu.VMEM((ECH, EB), jnp.int32),
            pltpu.VMEM((ECH, EB), jnp.int32),
            pltpu.VMEM((EB, H), jnp.float32),
            pltpu.VMEM((EB, H), jnp.float32),
            pltpu.VMEM((NPAD // NSUB, H), jnp.float32),
            pltpu.SemaphoreType.DMA,
            pltpu.SemaphoreType.DMA,
        ],
        compiler_params=_sc_params,
    )


_RB = 1000
_GRID = N // _RB


def _dinv_of(degt_block):
    return lax.rsqrt(jnp.sum(degt_block, axis=1, keepdims=True) + 1.0)


def _tc1_body(x_ref, w_ref, degt_ref, g_ref):
    h = jnp.dot(x_ref[...], w_ref[...], preferred_element_type=jnp.float32)
    g_ref[...] = h * _dinv_of(degt_ref[...])


def _tc_mid_body(a0_ref, a1_ref, g_ref, degt_ref, b_ref, w_ref, out_ref):
    dinv = _dinv_of(degt_ref[...])
    a = (a0_ref[...] + a1_ref[...] + g_ref[...]) * dinv + b_ref[...]
    r = jnp.maximum(a, 0.0)
    h = jnp.dot(r, w_ref[...], preferred_element_type=jnp.float32)
    out_ref[...] = h * dinv


def _tc_out_body(a0_ref, a1_ref, g_ref, degt_ref, b_ref, w_ref, b3_ref, out_ref):
    a = (a0_ref[...] + a1_ref[...] + g_ref[...]) * _dinv_of(degt_ref[...]) \
        + b_ref[...]
    r = jnp.maximum(a, 0.0)
    out_ref[...] = jnp.dot(r, w_ref[...],
                           preferred_element_type=jnp.float32) + b3_ref[...]


def _row_spec(width):
    return pl.BlockSpec((_RB, width), lambda i: (i, 0))


def _full_spec(shape):
    return pl.BlockSpec(shape, lambda i: tuple(0 for _ in shape))


def _tc1(x, W1, degt):
    return pl.pallas_call(
        _tc1_body,
        grid=(_GRID,),
        in_specs=[_row_spec(D_IN), _full_spec((D_IN, H)), _row_spec(NW)],
        out_specs=_row_spec(H),
        out_shape=jax.ShapeDtypeStruct((N, H), jnp.float32),
    )(x, W1, degt)


def _tc_mid(a0, a1, g, degt, b_row, W):
    return pl.pallas_call(
        _tc_mid_body,
        grid=(_GRID,),
        in_specs=[_row_spec(H), _row_spec(H), _row_spec(H), _row_spec(NW),
                  _full_spec((1, H)), _full_spec((H, H))],
        out_specs=_row_spec(H),
        out_shape=jax.ShapeDtypeStruct((N, H), jnp.float32),
    )(a0, a1, g, degt, b_row, W)


def _tc_out(a0, a1, g, degt, b_row, W3p, b3_row):
    return pl.pallas_call(
        _tc_out_body,
        grid=(_GRID,),
        in_specs=[_row_spec(H), _row_spec(H), _row_spec(H), _row_spec(NW),
                  _full_spec((1, H)), _full_spec((H, 8)), _full_spec((1, 8))],
        out_specs=_row_spec(8),
        out_shape=jax.ShapeDtypeStruct((N, 8), jnp.float32),
    )(a0, a1, g, degt, b_row, W3p, b3_row)


@jax.jit
def _run(x, src, dst, W1, b1, W2, b2, W3, b3):
    src = src.astype(jnp.int32)
    dst = dst.astype(jnp.int32)

    pad = EPAD - E
    srcp = jnp.concatenate([src, jnp.zeros((pad,), jnp.int32)]).reshape(NW, ECH, EB)
    dstp = jnp.concatenate(
        [dst, N + (jnp.arange(pad, dtype=jnp.int32) % (NPAD - N))]
    ).reshape(NW, ECH, EB)

    deg_parts = _make_sc_deg()(dstp)
    degt = deg_parts.T[:N]

    g1 = _tc1(x, W1, degt)

    agg_fn = _make_sc_agg()
    a1 = agg_fn(g1, srcp, dstp)
    g2 = _tc_mid(a1[0, :N], a1[1, :N], g1, degt, b1.reshape(1, H), W2)
    a2 = agg_fn(g2, srcp, dstp)
    out = _tc_out(a2[0, :N], a2[1, :N], g2, degt, b2.reshape(1, H),
                  jnp.zeros((H, 8), jnp.float32).at[:, :7].set(W3),
                  jnp.zeros((1, 8), jnp.float32).at[0, :7].set(b3))
    return out[:, :7]


def kernel(x, edge_index, W1, b1, W2, b2, W3, b3):
    return _run(x, edge_index[0], edge_index[1], W1, b1, W2, b2, W3, b3)


# speedup vs baseline: 41.8939x; 1.0699x over previous
import jax
import jax.numpy as jnp
from jax import lax
from jax.experimental import pallas as pl
from jax.experimental.pallas import tpu as pltpu
from jax.experimental.pallas import tpu_sc as plsc

N = 10000
E = 320000
D_IN = 128
H = 16
NCORES = 2
NSUB = 16
NW = NCORES * NSUB
EB = 80
ECH = 125
EPT = EB * ECH
RPS = N // NSUB

_mesh = plsc.VectorSubcoreMesh(core_axis_name="c", subcore_axis_name="s",
                               num_cores=NCORES, num_subcores=NSUB)

_sc_params = pltpu.CompilerParams(needs_layout_passes=False,
                                  use_tc_tiling_on_sc=False)


def _sc_deg_body(dst_hbm, deg_hbm, ldeg, dstb):
    c = lax.axis_index("c")
    s = lax.axis_index("s")
    t = c * NSUB + s

    zero16 = jnp.zeros((H,), jnp.float32)

    def _zero(i, _):
        ldeg[pl.ds(i * H, H)] = zero16
        return 0
    lax.fori_loop(0, N // H, _zero, 0)

    pltpu.sync_copy(dst_hbm.at[t], dstb)

    ones16 = jnp.ones((H,), jnp.float32)

    def _hist(r, _):
        for k in range(EB // H):
            idx = dstb[r, pl.ds(k * H, H)]
            plsc.addupdate_scatter(ldeg, [idx], ones16)
        return 0
    lax.fori_loop(0, ECH, _hist, 0)

    pltpu.sync_copy(ldeg, deg_hbm.at[t])


def _make_sc_deg():
    return pl.kernel(
        _sc_deg_body,
        out_type=jax.ShapeDtypeStruct((NW, N), jnp.float32),
        mesh=_mesh,
        scratch_types=[
            pltpu.VMEM((N,), jnp.float32),
            pltpu.VMEM((ECH, EB), jnp.int32),
        ],
        compiler_params=_sc_params,
    )


def _sc_agg_kernel(g_hbm, src_hbm, dst_hbm, agg_hbm, acc, srcb, dstb,
                   rows_a, rows_b, obuf, sem_a, sem_b):
    c = lax.axis_index("c")
    s = lax.axis_index("s")
    t = c * NSUB + s

    zero16 = jnp.zeros((H,), jnp.float32)

    def _zero(i, _):
        obuf[i] = zero16
        return 0
    lax.fori_loop(0, RPS, _zero, 0)
    pltpu.sync_copy(obuf, acc.at[pl.ds(RPS * s, RPS)])

    pltpu.sync_copy(src_hbm.at[t], srcb)
    pltpu.sync_copy(dst_hbm.at[t], dstb)
    plsc.subcore_barrier()

    def _wait(buf, sem):
        pltpu.make_async_copy(g_hbm.at[pl.ds(0, EB)], buf, sem).wait()

    pltpu.async_copy(g_hbm.at[srcb.at[0]], rows_a, sem_a)

    def _pair(i, _):
        j = 2 * i
        pltpu.async_copy(g_hbm.at[srcb.at[j + 1]], rows_b, sem_b)
        _wait(rows_a, sem_a)
        pltpu.sync_copy(rows_a, acc.at[dstb.at[j]], add=True)
        pltpu.async_copy(g_hbm.at[srcb.at[j + 2]], rows_a, sem_a)
        _wait(rows_b, sem_b)
        pltpu.sync_copy(rows_b, acc.at[dstb.at[j + 1]], add=True)
        return 0
    lax.fori_loop(0, (ECH - 1) // 2, _pair, 0)

    _wait(rows_a, sem_a)
    pltpu.sync_copy(rows_a, acc.at[dstb.at[ECH - 1]], add=True)

    plsc.subcore_barrier()
    pltpu.sync_copy(acc.at[pl.ds(RPS * s, RPS)], obuf)
    pltpu.sync_copy(obuf, agg_hbm.at[c].at[pl.ds(RPS * s, RPS)])


def _make_sc_agg():
    return pl.kernel(
        _sc_agg_kernel,
        out_type=jax.ShapeDtypeStruct((NCORES, N, H), jnp.float32),
        mesh=_mesh,
        scratch_types=[
            pltpu.VMEM_SHARED((N, H), jnp.float32),
            pltpu.VMEM((ECH, EB), jnp.int32),
            pltpu.VMEM((ECH, EB), jnp.int32),
            pltpu.VMEM((EB, H), jnp.float32),
            pltpu.VMEM((EB, H), jnp.float32),
            pltpu.VMEM((RPS, H), jnp.float32),
            pltpu.SemaphoreType.DMA,
            pltpu.SemaphoreType.DMA,
        ],
        compiler_params=_sc_params,
    )


_RB = 1000
_GRID = N // _RB


def _dinv_of(degt_block):
    return lax.rsqrt(jnp.sum(degt_block, axis=1, keepdims=True) + 1.0)


def _tc1_body(x_ref, w_ref, degt_ref, g_ref):
    h = jnp.dot(x_ref[...], w_ref[...], preferred_element_type=jnp.float32)
    g_ref[...] = h * _dinv_of(degt_ref[...])


def _tc_mid_body(a0_ref, a1_ref, g_ref, degt_ref, b_ref, w_ref, out_ref):
    dinv = _dinv_of(degt_ref[...])
    a = (a0_ref[0] + a1_ref[0] + g_ref[...]) * dinv + b_ref[...]
    r = jnp.maximum(a, 0.0)
    h = jnp.dot(r, w_ref[...], preferred_element_type=jnp.float32)
    out_ref[...] = h * dinv


def _tc_out_body(a0_ref, a1_ref, g_ref, degt_ref, b_ref, w_ref, b3_ref, out_ref):
    a = (a0_ref[0] + a1_ref[0] + g_ref[...]) * _dinv_of(degt_ref[...]) \
        + b_ref[...]
    r = jnp.maximum(a, 0.0)
    out_ref[...] = jnp.dot(r, w_ref[...],
                           preferred_element_type=jnp.float32) + b3_ref[...]


def _row_spec(width):
    return pl.BlockSpec((_RB, width), lambda i: (i, 0))


def _agg_spec(core):
    return pl.BlockSpec((1, _RB, H), lambda i, _c=core: (_c, i, 0))


def _full_spec(shape):
    return pl.BlockSpec(shape, lambda i: tuple(0 for _ in shape))


def _tc1(x, W1, degt):
    return pl.pallas_call(
        _tc1_body,
        grid=(_GRID,),
        in_specs=[_row_spec(D_IN), _full_spec((D_IN, H)), _row_spec(NW)],
        out_specs=_row_spec(H),
        out_shape=jax.ShapeDtypeStruct((N, H), jnp.float32),
    )(x, W1, degt)


def _tc_mid(agg, g, degt, b_row, W):
    return pl.pallas_call(
        _tc_mid_body,
        grid=(_GRID,),
        in_specs=[_agg_spec(0), _agg_spec(1), _row_spec(H), _row_spec(NW),
                  _full_spec((1, H)), _full_spec((H, H))],
        out_specs=_row_spec(H),
        out_shape=jax.ShapeDtypeStruct((N, H), jnp.float32),
    )(agg, agg, g, degt, b_row, W)


def _tc_out(agg, g, degt, b_row, W3p, b3_row):
    return pl.pallas_call(
        _tc_out_body,
        grid=(_GRID,),
        in_specs=[_agg_spec(0), _agg_spec(1), _row_spec(H), _row_spec(NW),
                  _full_spec((1, H)), _full_spec((H, 8)), _full_spec((1, 8))],
        out_specs=_row_spec(8),
        out_shape=jax.ShapeDtypeStruct((N, 8), jnp.float32),
    )(agg, agg, g, degt, b_row, W3p, b3_row)


@jax.jit
def _run(x, src, dst, W1, b1, W2, b2, W3, b3):
    srcp = src.astype(jnp.int32).reshape(NW, ECH, EB)
    dstp = dst.astype(jnp.int32).reshape(NW, ECH, EB)

    deg_parts = _make_sc_deg()(dstp)
    degt = deg_parts.T

    g1 = _tc1(x, W1, degt)

    agg_fn = _make_sc_agg()
    a1 = agg_fn(g1, srcp, dstp)
    g2 = _tc_mid(a1, g1, degt, b1.reshape(1, H), W2)
    a2 = agg_fn(g2, srcp, dstp)
    out = _tc_out(a2, g2, degt, b2.reshape(1, H),
                  jnp.zeros((H, 8), jnp.float32).at[:, :7].set(W3),
                  jnp.zeros((1, 8), jnp.float32).at[0, :7].set(b3))
    return out[:, :7]


def kernel(x, edge_index, W1, b1, W2, b2, W3, b3):
    return _run(x, edge_index[0], edge_index[1], W1, b1, W2, b2, W3, b3)


# speedup vs baseline: 52.6556x; 1.2569x over previous
import jax
import jax.numpy as jnp
from jax import lax
from jax.experimental import pallas as pl
from jax.experimental.pallas import tpu as pltpu
from jax.experimental.pallas import tpu_sc as plsc

N = 10000
E = 320000
D_IN = 128
H = 16
NCORES = 2
NSUB = 16
NW = NCORES * NSUB
EB = 80
ECH = 125
EPT = EB * ECH
RPS = N // NSUB

_mesh = plsc.VectorSubcoreMesh(core_axis_name="c", subcore_axis_name="s",
                               num_cores=NCORES, num_subcores=NSUB)

_sc_params = pltpu.CompilerParams(needs_layout_passes=False,
                                  use_tc_tiling_on_sc=False)


def _sc_deg_body(dst_hbm, deg_hbm, ldeg, dstb):
    c = lax.axis_index("c")
    s = lax.axis_index("s")
    t = c * NSUB + s

    zero16 = jnp.zeros((H,), jnp.float32)

    def _zero(i, _):
        ldeg[pl.ds(i * H, H)] = zero16
        return 0
    lax.fori_loop(0, N // H, _zero, 0)

    pltpu.sync_copy(dst_hbm.at[t], dstb)

    ones16 = jnp.ones((H,), jnp.float32)

    def _hist(r, _):
        for k in range(EB // H):
            idx = dstb[r, pl.ds(k * H, H)]
            plsc.addupdate_scatter(ldeg, [idx], ones16)
        return 0
    lax.fori_loop(0, ECH, _hist, 0)

    pltpu.sync_copy(ldeg, deg_hbm.at[t])


def _make_sc_deg():
    return pl.kernel(
        _sc_deg_body,
        out_type=jax.ShapeDtypeStruct((NW, N), jnp.float32),
        mesh=_mesh,
        scratch_types=[
            pltpu.VMEM((N,), jnp.float32),
            pltpu.VMEM((ECH, EB), jnp.int32),
        ],
        compiler_params=_sc_params,
    )


GRP = 5
NG = ECH // GRP


def _sc_agg_kernel(g_hbm, src_hbm, dst_hbm, agg_hbm, acc, srcb, dstb,
                   bufs_a, bufs_b, obuf, gsem_a, gsem_b, ssem_a, ssem_b,
                   isem):
    c = lax.axis_index("c")
    s = lax.axis_index("s")
    t = c * NSUB + s

    cp_src = pltpu.async_copy(src_hbm.at[t], srcb, isem)
    cp_dst = pltpu.async_copy(dst_hbm.at[t], dstb, isem)

    zero16 = jnp.zeros((H,), jnp.float32)

    def _zero(i, _):
        obuf[i] = zero16
        return 0
    lax.fori_loop(0, RPS, _zero, 0)
    pltpu.sync_copy(obuf, acc.at[pl.ds(RPS * s, RPS)])

    cp_src.wait()
    cp_dst.wait()
    plsc.subcore_barrier()

    def _drain(sem):
        pltpu.make_async_copy(g_hbm.at[pl.ds(0, EB)], bufs_a.at[0], sem).wait()

    def _gathers(grp, bufs, gsem):
        for k in range(GRP):
            pltpu.async_copy(g_hbm.at[srcb.at[grp * GRP + k]], bufs.at[k], gsem)

    def _scatters(grp, bufs, ssem):
        for k in range(GRP):
            pltpu.async_copy(bufs.at[k], acc.at[dstb.at[grp * GRP + k]], ssem,
                             add=True)

    _gathers(0, bufs_a, gsem_a)

    def _step(i, _):
        for _k in range(GRP):
            _drain(gsem_a)

        @pl.when(i > 0)
        def _():
            for _k in range(GRP):
                _drain(ssem_b)

        _gathers(2 * i + 1, bufs_b, gsem_b)
        _scatters(2 * i, bufs_a, ssem_a)

        for _k in range(GRP):
            _drain(gsem_b)
        for _k in range(GRP):
            _drain(ssem_a)
        _gathers(2 * i + 2, bufs_a, gsem_a)
        _scatters(2 * i + 1, bufs_b, ssem_b)
        return 0
    lax.fori_loop(0, (NG - 1) // 2, _step, 0)

    for _k in range(GRP):
        _drain(gsem_a)
    for _k in range(GRP):
        _drain(ssem_b)
    _scatters(NG - 1, bufs_a, ssem_a)
    for _k in range(GRP):
        _drain(ssem_a)

    plsc.subcore_barrier()
    pltpu.sync_copy(acc.at[pl.ds(RPS * s, RPS)], obuf)
    pltpu.sync_copy(obuf, agg_hbm.at[c].at[pl.ds(RPS * s, RPS)])


def _make_sc_agg():
    return pl.kernel(
        _sc_agg_kernel,
        out_type=jax.ShapeDtypeStruct((NCORES, N, H), jnp.float32),
        mesh=_mesh,
        scratch_types=[
            pltpu.VMEM_SHARED((N, H), jnp.float32),
            pltpu.VMEM((ECH, EB), jnp.int32),
            pltpu.VMEM((ECH, EB), jnp.int32),
            pltpu.VMEM((GRP, EB, H), jnp.float32),
            pltpu.VMEM((GRP, EB, H), jnp.float32),
            pltpu.VMEM((RPS, H), jnp.float32),
            pltpu.SemaphoreType.DMA,
            pltpu.SemaphoreType.DMA,
            pltpu.SemaphoreType.DMA,
            pltpu.SemaphoreType.DMA,
            pltpu.SemaphoreType.DMA,
        ],
        compiler_params=_sc_params,
    )


_RB = 1000
_GRID = N // _RB


def _dinv_of(degt_block):
    return lax.rsqrt(jnp.sum(degt_block, axis=1, keepdims=True) + 1.0)


def _tc1_body(x_ref, w_ref, degt_ref, g_ref):
    h = jnp.dot(x_ref[...], w_ref[...], preferred_element_type=jnp.float32)
    g_ref[...] = h * _dinv_of(degt_ref[...])


def _tc_mid_body(a0_ref, a1_ref, g_ref, degt_ref, b_ref, w_ref, out_ref):
    dinv = _dinv_of(degt_ref[...])
    a = (a0_ref[0] + a1_ref[0] + g_ref[...]) * dinv + b_ref[...]
    r = jnp.maximum(a, 0.0)
    h = jnp.dot(r, w_ref[...], preferred_element_type=jnp.float32)
    out_ref[...] = h * dinv


def _tc_out_body(a0_ref, a1_ref, g_ref, degt_ref, b_ref, w_ref, b3_ref, out_ref):
    a = (a0_ref[0] + a1_ref[0] + g_ref[...]) * _dinv_of(degt_ref[...]) \
        + b_ref[...]
    r = jnp.maximum(a, 0.0)
    out_ref[...] = jnp.dot(r, w_ref[...],
                           preferred_element_type=jnp.float32) + b3_ref[...]


def _row_spec(width):
    return pl.BlockSpec((_RB, width), lambda i: (i, 0))


def _agg_spec(core):
    return pl.BlockSpec((1, _RB, H), lambda i, _c=core: (_c, i, 0))


def _full_spec(shape):
    return pl.BlockSpec(shape, lambda i: tuple(0 for _ in shape))


def _tc1(x, W1, degt):
    return pl.pallas_call(
        _tc1_body,
        grid=(_GRID,),
        in_specs=[_row_spec(D_IN), _full_spec((D_IN, H)), _row_spec(NW)],
        out_specs=_row_spec(H),
        out_shape=jax.ShapeDtypeStruct((N, H), jnp.float32),
    )(x, W1, degt)


def _tc_mid(agg, g, degt, b_row, W):
    return pl.pallas_call(
        _tc_mid_body,
        grid=(_GRID,),
        in_specs=[_agg_spec(0), _agg_spec(1), _row_spec(H), _row_spec(NW),
                  _full_spec((1, H)), _full_spec((H, H))],
        out_specs=_row_spec(H),
        out_shape=jax.ShapeDtypeStruct((N, H), jnp.float32),
    )(agg, agg, g, degt, b_row, W)


def _tc_out(agg, g, degt, b_row, W3p, b3_row):
    return pl.pallas_call(
        _tc_out_body,
        grid=(_GRID,),
        in_specs=[_agg_spec(0), _agg_spec(1), _row_spec(H), _row_spec(NW),
                  _full_spec((1, H)), _full_spec((H, 8)), _full_spec((1, 8))],
        out_specs=_row_spec(8),
        out_shape=jax.ShapeDtypeStruct((N, 8), jnp.float32),
    )(agg, agg, g, degt, b_row, W3p, b3_row)


@jax.jit
def _run(x, src, dst, W1, b1, W2, b2, W3, b3):
    srcp = src.astype(jnp.int32).reshape(NW, ECH, EB)
    dstp = dst.astype(jnp.int32).reshape(NW, ECH, EB)

    deg_parts = _make_sc_deg()(dstp)
    degt = deg_parts.T

    g1 = _tc1(x, W1, degt)

    agg_fn = _make_sc_agg()
    a1 = agg_fn(g1, srcp, dstp)
    g2 = _tc_mid(a1, g1, degt, b1.reshape(1, H), W2)
    a2 = agg_fn(g2, srcp, dstp)
    out = _tc_out(a2, g2, degt, b2.reshape(1, H),
                  jnp.zeros((H, 8), jnp.float32).at[:, :7].set(W3),
                  jnp.zeros((1, 8), jnp.float32).at[0, :7].set(b3))
    return out[:, :7]


def kernel(x, edge_index, W1, b1, W2, b2, W3, b3):
    return _run(x, edge_index[0], edge_index[1], W1, b1, W2, b2, W3, b3)
